# Initial kernel scaffold; baseline (speedup 1.0000x reference)
import jax
import jax.numpy as jnp
from jax.experimental import pallas as pl


def kernel(partial_charges, Z, ns, idx_m, Rij, idx_i, idx_j, is_film, r0_table):
    raise NotImplementedError("write your pallas kernel here")


# speedup vs baseline: 48.8720x; 48.8720x over previous
import functools

import jax
import jax.numpy as jnp
from jax import lax
from jax.experimental import pallas as pl
from jax.experimental.pallas import tpu as pltpu
from jax.experimental.pallas import tpu_sc as plsc

N_ATOMS = 100000
N_EDGES = 6400000
N_MOL = 512
KE = 14.3996
LN2 = 0.6931471805599453
LN5 = 1.6094379124341003

CHUNK = 1024
SUB = 128
NSUB = CHUNK // SUB
NCHUNKS = N_EDGES // CHUNK
NW = 32
GROUPS = CHUNK // 16


def _full(v):
    return jnp.full((16,), v, jnp.int32)


def _body(tab, ii, jj, rij, lr0, out, ii_v, jj_v, rij_v, rows_i, rows_j,
          lr0_v, hist_v, sem):
    wid = lax.axis_index("s") * 2 + lax.axis_index("c")

    pltpu.async_copy(lr0, lr0_v, sem).wait()

    zeros = jnp.zeros((16,), jnp.float32)
    for z in range(N_MOL // 16):
        hist_v[pl.ds(z * 16, 16)] = zeros

    lane = lax.iota(jnp.int32, 16)

    def group_body(g, carry):
        r = g * 16 + lane
        qi = plsc.load_gather(rows_i, [r, _full(0)])
        nsi = plsc.load_gather(rows_i, [r, _full(1)])
        ci = plsc.load_gather(rows_i, [r, _full(2)])
        qj = plsc.load_gather(rows_j, [r, _full(0)])
        nsj = plsc.load_gather(rows_j, [r, _full(1)])
        cb = plsc.load_gather(rows_j, [r, _full(3)])
        x = plsc.load_gather(rij_v, [r, _full(0)])
        y = plsc.load_gather(rij_v, [r, _full(1)])
        z = plsc.load_gather(rij_v, [r, _full(2)])
        d2 = x * x + y * y + z * z

        cii = ci.astype(jnp.int32)
        m = cii >> 10
        av = cii & 1023
        ridx = av + cb.astype(jnp.int32)
        lr = plsc.load_gather(lr0_v, [ridx])

        n = nsi + nsj * 0.5
        bits = lax.bitcast_convert_type(d2, jnp.int32)
        e = ((bits >> 23) & 0xFF) - 127
        mbits = (bits & 0x7FFFFF) | 0x3F800000
        mf = lax.bitcast_convert_type(mbits, jnp.float32)
        rr = (mf - 1.0) / (mf + 1.0)
        r2 = rr * rr
        poly = rr * (2.0 + r2 * (2.0 / 3.0 + r2 * (2.0 / 5.0
                     + r2 * (2.0 / 7.0 + r2 * (2.0 / 9.0)))))
        lnd = 0.5 * (e.astype(jnp.float32) * LN2 + poly)
        u = (n - 1.0) * lr
        p1 = jnp.exp(u - n * lnd)
        p2 = jnp.exp(u - n * LN5)
        coef = (0.5 * KE) * jnp.abs(qi * qj) / n
        pot = coef * (p1 - p2)
        pot = jnp.where(d2 <= 25.0, pot, 0.0)
        plsc.addupdate_scatter(hist_v, [m], pot)
        return carry

    def chunk_body(k, carry):
        g = wid + k * NW
        c1 = pltpu.async_copy(ii.at[g], ii_v, sem)
        c2 = pltpu.async_copy(jj.at[g], jj_v, sem)
        c3 = pltpu.async_copy(rij.at[g], rij_v, sem)
        c1.wait()
        c2.wait()
        descs = []
        for s in range(NSUB):
            descs.append(pltpu.async_copy(
                tab.at[ii_v.at[s]], rows_i.at[pl.ds(s * SUB, SUB), :], sem))
            descs.append(pltpu.async_copy(
                tab.at[jj_v.at[s]], rows_j.at[pl.ds(s * SUB, SUB), :], sem))
        for d in descs:
            d.wait()
        c3.wait()
        lax.fori_loop(0, GROUPS, group_body, 0, unroll=2)
        return carry

    nk = jnp.where(wid < (NCHUNKS % NW), NCHUNKS // NW + 1, NCHUNKS // NW)
    lax.fori_loop(0, nk, chunk_body, 0)

    pltpu.async_copy(hist_v, out.at[wid], sem).wait()


def _partials(partial_charges, Z, ns, idx_m, Rij, idx_i, idx_j, is_film,
              r0_table):
    q = jnp.squeeze(partial_charges, -1).astype(jnp.float32)
    Zi = Z.astype(jnp.int32)
    film = is_film.astype(jnp.int32)
    mol = idx_m.astype(jnp.int32)
    ci = mol * 1024 + film * 200 + Zi * 10
    cb = film * 100 + Zi
    cols = jnp.stack([q, ns.astype(jnp.float32),
                      ci.astype(jnp.float32), cb.astype(jnp.float32)], axis=-1)
    tab = jnp.pad(cols, ((0, 0), (0, 12)))
    lr0 = jnp.log(r0_table.astype(jnp.float32)).reshape(-1)
    ii = idx_i.astype(jnp.int32).reshape(NCHUNKS, NSUB, SUB)
    jj = idx_j.astype(jnp.int32).reshape(NCHUNKS, NSUB, SUB)
    rij = Rij.astype(jnp.float32).reshape(NCHUNKS, CHUNK, 3)

    mesh = plsc.VectorSubcoreMesh(core_axis_name="c", subcore_axis_name="s")
    run = functools.partial(
        pl.kernel,
        mesh=mesh,
        out_type=jax.ShapeDtypeStruct((NW, N_MOL), jnp.float32),
        compiler_params=pltpu.CompilerParams(
            needs_layout_passes=False, use_tc_tiling_on_sc=False),
        scratch_types=[
            pltpu.VMEM((NSUB, SUB), jnp.int32),
            pltpu.VMEM((NSUB, SUB), jnp.int32),
            pltpu.VMEM((CHUNK, 3), jnp.float32),
            pltpu.VMEM((CHUNK, 16), jnp.float32),
            pltpu.VMEM((CHUNK, 16), jnp.float32),
            pltpu.VMEM((400,), jnp.float32),
            pltpu.VMEM((N_MOL,), jnp.float32),
            pltpu.SemaphoreType.DMA,
        ],
    )(_body)
    return run(tab, ii, jj, rij, lr0)


def kernel(partial_charges, Z, ns, idx_m, Rij, idx_i, idx_j, is_film, r0_table):
    partials = _partials(partial_charges, Z, ns, idx_m, Rij, idx_i, idx_j,
                         is_film, r0_table)
    return partials.sum(axis=0)


# speedup vs baseline: 58.2650x; 1.1922x over previous
import functools

import jax
import jax.numpy as jnp
from jax import lax
from jax.experimental import pallas as pl
from jax.experimental.pallas import tpu as pltpu
from jax.experimental.pallas import tpu_sc as plsc

N_ATOMS = 100000
N_EDGES = 6400000
N_MOL = 512
KE = 14.3996
LN2 = 0.6931471805599453
LN5 = 1.6094379124341003

CHUNK = 1024
SUB = 128
NSUB = CHUNK // SUB
NCHUNKS = N_EDGES // CHUNK
NW = 32
GROUPS = CHUNK // 16


def _full(v):
    return jnp.full((16,), v, jnp.int32)


def _body(tab, ii, jj, rij, lr0, out, ii_v, jj_v, rij_v, rows_i, rows_j,
          lr0_v, hist_v, sem):
    wid = lax.axis_index("s") * 2 + lax.axis_index("c")

    pltpu.async_copy(lr0, lr0_v, sem).wait()

    zeros = jnp.zeros((16,), jnp.float32)
    for z in range(N_MOL // 16):
        hist_v[pl.ds(z * 16, 16)] = zeros

    lane = lax.iota(jnp.int32, 16)

    def group_body(g, carry):
        r = g * 16 + lane
        e3 = r * 3
        qi = plsc.load_gather(rows_i, [r, _full(0)])
        nsi = plsc.load_gather(rows_i, [r, _full(1)])
        ci = plsc.load_gather(rows_i, [r, _full(2)])
        qj = plsc.load_gather(rows_j, [r, _full(0)])
        nsj = plsc.load_gather(rows_j, [r, _full(1)])
        cb = plsc.load_gather(rows_j, [r, _full(3)])
        x = plsc.load_gather(rij_v, [e3])
        y = plsc.load_gather(rij_v, [e3 + 1])
        z = plsc.load_gather(rij_v, [e3 + 2])
        d2 = x * x + y * y + z * z

        cii = ci.astype(jnp.int32)
        m = cii >> 10
        av = cii & 1023
        ridx = av + cb.astype(jnp.int32)
        lr = plsc.load_gather(lr0_v, [ridx])

        n = nsi + nsj * 0.5
        bits = lax.bitcast_convert_type(d2, jnp.int32)
        e = ((bits >> 23) & 0xFF) - 127
        mbits = (bits & 0x7FFFFF) | 0x3F800000
        mf = lax.bitcast_convert_type(mbits, jnp.float32)
        rr = (mf - 1.0) / (mf + 1.0)
        r2 = rr * rr
        poly = rr * (2.0 + r2 * (2.0 / 3.0 + r2 * (2.0 / 5.0
                     + r2 * (2.0 / 7.0 + r2 * (2.0 / 9.0)))))
        lnd = 0.5 * (e.astype(jnp.float32) * LN2 + poly)
        u = (n - 1.0) * lr
        p1 = jnp.exp(u - n * lnd)
        p2 = jnp.exp(u - n * LN5)
        coef = (0.5 * KE) * jnp.abs(qi * qj) / n
        pot = coef * (p1 - p2)
        pot = jnp.where(d2 <= 25.0, pot, 0.0)
        plsc.addupdate_scatter(hist_v, [m], pot)
        return carry

    def chunk_body(k, carry):
        g = wid + k * NW
        c1 = pltpu.async_copy(ii.at[pl.ds(g * CHUNK, CHUNK)], ii_v, sem)
        c2 = pltpu.async_copy(jj.at[pl.ds(g * CHUNK, CHUNK)], jj_v, sem)
        c3 = pltpu.async_copy(rij.at[pl.ds(g * CHUNK * 3, CHUNK * 3)], rij_v, sem)
        c1.wait()
        c2.wait()
        descs = []
        for s in range(NSUB):
            descs.append(pltpu.async_copy(
                tab.at[ii_v.at[pl.ds(s * SUB, SUB)]],
                rows_i.at[pl.ds(s * SUB, SUB), :], sem))
            descs.append(pltpu.async_copy(
                tab.at[jj_v.at[pl.ds(s * SUB, SUB)]],
                rows_j.at[pl.ds(s * SUB, SUB), :], sem))
        for d in descs:
            d.wait()
        c3.wait()
        lax.fori_loop(0, GROUPS, group_body, 0, unroll=2)
        return carry

    nk = jnp.where(wid < (NCHUNKS % NW), NCHUNKS // NW + 1, NCHUNKS // NW)
    lax.fori_loop(0, nk, chunk_body, 0)

    pltpu.async_copy(hist_v, out.at[wid], sem).wait()


def _partials(partial_charges, Z, ns, idx_m, Rij, idx_i, idx_j, is_film,
              r0_table):
    q = jnp.squeeze(partial_charges, -1).astype(jnp.float32)
    Zi = Z.astype(jnp.int32)
    film = is_film.astype(jnp.int32)
    mol = idx_m.astype(jnp.int32)
    ci = mol * 1024 + film * 200 + Zi * 10
    cb = film * 100 + Zi
    cols = jnp.stack([q, ns.astype(jnp.float32),
                      ci.astype(jnp.float32), cb.astype(jnp.float32)], axis=-1)
    tab = jnp.pad(cols, ((0, 0), (0, 12)))
    lr0 = jnp.log(r0_table.astype(jnp.float32)).reshape(-1)
    ii = idx_i.astype(jnp.int32)
    jj = idx_j.astype(jnp.int32)
    rij = Rij.astype(jnp.float32).reshape(N_EDGES * 3)

    mesh = plsc.VectorSubcoreMesh(core_axis_name="c", subcore_axis_name="s")
    run = functools.partial(
        pl.kernel,
        mesh=mesh,
        out_type=jax.ShapeDtypeStruct((NW, N_MOL), jnp.float32),
        compiler_params=pltpu.CompilerParams(
            needs_layout_passes=False, use_tc_tiling_on_sc=False),
        scratch_types=[
            pltpu.VMEM((CHUNK,), jnp.int32),
            pltpu.VMEM((CHUNK,), jnp.int32),
            pltpu.VMEM((CHUNK * 3,), jnp.float32),
            pltpu.VMEM((CHUNK, 16), jnp.float32),
            pltpu.VMEM((CHUNK, 16), jnp.float32),
            pltpu.VMEM((400,), jnp.float32),
            pltpu.VMEM((N_MOL,), jnp.float32),
            pltpu.SemaphoreType.DMA,
        ],
    )(_body)
    return run(tab, ii, jj, rij, lr0)


def kernel(partial_charges, Z, ns, idx_m, Rij, idx_i, idx_j, is_film, r0_table):
    partials = _partials(partial_charges, Z, ns, idx_m, Rij, idx_i, idx_j,
                         is_film, r0_table)
    return partials.sum(axis=0)


# speedup vs baseline: 97.9497x; 1.6811x over previous
import functools

import jax
import jax.numpy as jnp
from jax import lax
from jax.experimental import pallas as pl
from jax.experimental.pallas import tpu as pltpu
from jax.experimental.pallas import tpu_sc as plsc

N_ATOMS = 100000
N_EDGES = 6400000
N_MOL = 512
KE = 14.3996
LN5 = 1.6094379124341003

CHUNK = 1024
SUB = 128
NSUB = CHUNK // SUB
NCHUNKS = N_EDGES // CHUNK
NW = 32
GROUPS = CHUNK // 16

BT = 10240


def _full(v):
    return jnp.full((16,), v, jnp.int32)


def _lnd_body(rij_ref, out_ref):
    b = rij_ref[...]
    d2 = jnp.sum(b * b, axis=1)
    out_ref[...] = 0.5 * jnp.log(d2)


def _lnd(rij):
    return pl.pallas_call(
        _lnd_body,
        grid=(N_EDGES // BT,),
        in_specs=[pl.BlockSpec((BT, 3), lambda i: (i, 0))],
        out_specs=pl.BlockSpec((BT,), lambda i: (i,)),
        out_shape=jax.ShapeDtypeStruct((N_EDGES,), jnp.float32),
    )(rij)


def _body(tab, ii, jj, lnd, lr0, out, ii_v, jj_v, lnd_v, rows_i, rows_j,
          lr0_v, hist_v, sem):
    wid = lax.axis_index("s") * 2 + lax.axis_index("c")

    pltpu.async_copy(lr0, lr0_v, sem).wait()

    zeros = jnp.zeros((16,), jnp.float32)
    for z in range(N_MOL // 16):
        hist_v[pl.ds(z * 16, 16)] = zeros

    lane = lax.iota(jnp.int32, 16)

    def group_body(g, carry):
        r = g * 16 + lane
        qi = plsc.load_gather(rows_i, [r, _full(0)])
        nsi = plsc.load_gather(rows_i, [r, _full(1)])
        ci = plsc.load_gather(rows_i, [r, _full(2)])
        qj = plsc.load_gather(rows_j, [r, _full(0)])
        nsj = plsc.load_gather(rows_j, [r, _full(1)])
        cb = plsc.load_gather(rows_j, [r, _full(3)])
        lnd16 = lnd_v[pl.ds(g * 16, 16)]

        cii = ci.astype(jnp.int32)
        m = cii >> 10
        av = cii & 1023
        ridx = av + cb.astype(jnp.int32)
        lr = plsc.load_gather(lr0_v, [ridx])

        n = nsi + nsj * 0.5
        u = (n - 1.0) * lr
        p1 = jnp.exp(u - n * lnd16)
        p2 = jnp.exp(u - n * LN5)
        coef = (0.5 * KE) * jnp.abs(qi * qj) / n
        pot = coef * (p1 - p2)
        pot = jnp.where(lnd16 <= LN5, pot, 0.0)
        plsc.addupdate_scatter(hist_v, [m], pot)
        return carry

    def chunk_body(k, carry):
        g = wid + k * NW
        c1 = pltpu.async_copy(ii.at[pl.ds(g * CHUNK, CHUNK)], ii_v, sem)
        c2 = pltpu.async_copy(jj.at[pl.ds(g * CHUNK, CHUNK)], jj_v, sem)
        c3 = pltpu.async_copy(lnd.at[pl.ds(g * CHUNK, CHUNK)], lnd_v, sem)
        c1.wait()
        c2.wait()
        descs = []
        for s in range(NSUB):
            descs.append(pltpu.async_copy(
                tab.at[ii_v.at[pl.ds(s * SUB, SUB)]],
                rows_i.at[pl.ds(s * SUB, SUB), :], sem))
            descs.append(pltpu.async_copy(
                tab.at[jj_v.at[pl.ds(s * SUB, SUB)]],
                rows_j.at[pl.ds(s * SUB, SUB), :], sem))
        for d in descs:
            d.wait()
        c3.wait()
        lax.fori_loop(0, GROUPS, group_body, 0, unroll=2)
        return carry

    nk = jnp.where(wid < (NCHUNKS % NW), NCHUNKS // NW + 1, NCHUNKS // NW)
    lax.fori_loop(0, nk, chunk_body, 0)

    pltpu.async_copy(hist_v, out.at[wid], sem).wait()


def _partials(partial_charges, Z, ns, idx_m, Rij, idx_i, idx_j, is_film,
              r0_table):
    q = jnp.squeeze(partial_charges, -1).astype(jnp.float32)
    Zi = Z.astype(jnp.int32)
    film = is_film.astype(jnp.int32)
    mol = idx_m.astype(jnp.int32)
    ci = mol * 1024 + film * 200 + Zi * 10
    cb = film * 100 + Zi
    cols = jnp.stack([q, ns.astype(jnp.float32),
                      ci.astype(jnp.float32), cb.astype(jnp.float32)], axis=-1)
    tab = jnp.pad(cols, ((0, 0), (0, 12)))
    lr0 = jnp.log(r0_table.astype(jnp.float32)).reshape(-1)
    ii = idx_i.astype(jnp.int32)
    jj = idx_j.astype(jnp.int32)
    lnd = _lnd(Rij.astype(jnp.float32))

    mesh = plsc.VectorSubcoreMesh(core_axis_name="c", subcore_axis_name="s")
    run = functools.partial(
        pl.kernel,
        mesh=mesh,
        out_type=jax.ShapeDtypeStruct((NW, N_MOL), jnp.float32),
        compiler_params=pltpu.CompilerParams(
            needs_layout_passes=False, use_tc_tiling_on_sc=False),
        scratch_types=[
            pltpu.VMEM((CHUNK,), jnp.int32),
            pltpu.VMEM((CHUNK,), jnp.int32),
            pltpu.VMEM((CHUNK,), jnp.float32),
            pltpu.VMEM((CHUNK, 16), jnp.float32),
            pltpu.VMEM((CHUNK, 16), jnp.float32),
            pltpu.VMEM((400,), jnp.float32),
            pltpu.VMEM((N_MOL,), jnp.float32),
            pltpu.SemaphoreType.DMA,
        ],
    )(_body)
    return run(tab, ii, jj, lnd, lr0)


def kernel(partial_charges, Z, ns, idx_m, Rij, idx_i, idx_j, is_film, r0_table):
    partials = _partials(partial_charges, Z, ns, idx_m, Rij, idx_i, idx_j,
                         is_film, r0_table)
    return partials.sum(axis=0)
